# baseline (speedup 1.0000x reference)
import jax
import jax.numpy as jnp
from jax import lax
from jax.experimental import pallas as pl
from jax.experimental.pallas import tpu as pltpu
from jax.experimental.pallas import tpu_sc as plsc

B = 16384
D = 32
NC = 2
NS = 16
NW = NC * NS
BPW = B // NW
CHUNK = 128
NCHUNK = BPW // CHUNK

BLK = 2048


def _sc_gather(cust_ids, prod_ids, cust_emb, prod_emb, cust_bias, prod_bias,
               ce_out, pe_out, cb_out, pb_out,
               cidx_v, pidx_v, ce_v, pe_v, cb_v, pb_v, sem):
    wid = lax.axis_index("s") * NC + lax.axis_index("c")
    base = wid * BPW
    pltpu.sync_copy(cust_ids.at[pl.ds(base, BPW)], cidx_v)
    pltpu.sync_copy(prod_ids.at[pl.ds(base, BPW)], pidx_v)

    def chunk_body(c, _):
        def issue(g, _):
            idc = cidx_v[pl.ds(c * CHUNK + g * 16, 16)]
            idp = pidx_v[pl.ds(c * CHUNK + g * 16, 16)]
            for l in range(16):
                j = g * 16 + l
                cid = idc[l]
                pid = idp[l]
                pltpu.async_copy(cust_emb.at[pl.ds(cid, 1)],
                                 ce_v.at[pl.ds(j, 1)], sem)
                pltpu.async_copy(prod_emb.at[pl.ds(pid, 1)],
                                 pe_v.at[pl.ds(j, 1)], sem)
                pltpu.async_copy(cust_bias.at[pl.ds(cid, 1)],
                                 cb_v.at[pl.ds(j, 1)], sem)
                pltpu.async_copy(prod_bias.at[pl.ds(pid, 1)],
                                 pb_v.at[pl.ds(j, 1)], sem)
            return 0

        lax.fori_loop(0, CHUNK // 16, issue, 0)
        pltpu.make_async_copy(cust_emb.at[pl.ds(0, CHUNK)], ce_v, sem).wait()
        pltpu.make_async_copy(prod_emb.at[pl.ds(0, CHUNK)], pe_v, sem).wait()
        pltpu.make_async_copy(cust_bias.at[pl.ds(0, CHUNK)], cb_v, sem).wait()
        pltpu.make_async_copy(prod_bias.at[pl.ds(0, CHUNK)], pb_v, sem).wait()
        out = base + c * CHUNK
        pltpu.sync_copy(ce_v, ce_out.at[pl.ds(out, CHUNK)])
        pltpu.sync_copy(pe_v, pe_out.at[pl.ds(out, CHUNK)])
        pltpu.sync_copy(cb_v, cb_out.at[pl.ds(out, CHUNK)])
        pltpu.sync_copy(pb_v, pb_out.at[pl.ds(out, CHUNK)])
        return 0

    lax.fori_loop(0, NCHUNK, chunk_body, 0)


def _dense_body(ce_ref, pe_ref, cb_ref, pb_ref, w1c_ref, w1p_ref, b1_ref,
                w2_ref, b2_ref, w3_ref, const_ref, out_ref):
    ce = ce_ref[...]
    pe = pe_ref[...]
    mf = jnp.sum(ce * pe, axis=1, keepdims=True)
    h1 = jnp.maximum(
        jnp.dot(ce, w1c_ref[...], preferred_element_type=jnp.float32)
        + jnp.dot(pe, w1p_ref[...], preferred_element_type=jnp.float32)
        + b1_ref[...], 0.0)
    h2 = jnp.maximum(
        jnp.dot(h1, w2_ref[...], preferred_element_type=jnp.float32)
        + b2_ref[...], 0.0)
    mlp = jnp.sum(h2 * w3_ref[...], axis=1, keepdims=True)
    logit = (0.6 * mf + 0.4 * mlp + cb_ref[...] + pb_ref[...]
             + const_ref[...])
    out_ref[...] = jax.nn.sigmoid(logit)


def kernel(customer_ids, product_ids, cust_emb, prod_emb, cust_bias,
           prod_bias, global_bias, W1, b1, W2, b2, W3, b3):
    cids = customer_ids.astype(jnp.int32)
    pids = product_ids.astype(jnp.int32)

    mesh = plsc.VectorSubcoreMesh(
        core_axis_name="c", subcore_axis_name="s",
        num_cores=NC, num_subcores=NS)
    sc_call = pl.kernel(
        _sc_gather,
        out_type=[
            jax.ShapeDtypeStruct((B, D), jnp.float32),
            jax.ShapeDtypeStruct((B, D), jnp.float32),
            jax.ShapeDtypeStruct((B, 1), jnp.float32),
            jax.ShapeDtypeStruct((B, 1), jnp.float32),
        ],
        mesh=mesh,
        scratch_types=[
            pltpu.VMEM((BPW,), jnp.int32),
            pltpu.VMEM((BPW,), jnp.int32),
            pltpu.VMEM((CHUNK, D), jnp.float32),
            pltpu.VMEM((CHUNK, D), jnp.float32),
            pltpu.VMEM((CHUNK, 1), jnp.float32),
            pltpu.VMEM((CHUNK, 1), jnp.float32),
            pltpu.SemaphoreType.DMA,
        ],
        compiler_params=pltpu.CompilerParams(use_tc_tiling_on_sc=True),
    )
    ce, pe, cb, pb = sc_call(cids, pids, cust_emb, prod_emb,
                             cust_bias, prod_bias)

    w1c = W1[:D, :]
    w1p = W1[D:, :]
    const = (0.4 * b3 + global_bias).reshape(1, 1)

    grid = (B // BLK,)
    out = pl.pallas_call(
        _dense_body,
        grid=grid,
        in_specs=[
            pl.BlockSpec((BLK, D), lambda i: (i, 0)),
            pl.BlockSpec((BLK, D), lambda i: (i, 0)),
            pl.BlockSpec((BLK, 1), lambda i: (i, 0)),
            pl.BlockSpec((BLK, 1), lambda i: (i, 0)),
            pl.BlockSpec((D, 64), lambda i: (0, 0)),
            pl.BlockSpec((D, 64), lambda i: (0, 0)),
            pl.BlockSpec((1, 64), lambda i: (0, 0)),
            pl.BlockSpec((64, 32), lambda i: (0, 0)),
            pl.BlockSpec((1, 32), lambda i: (0, 0)),
            pl.BlockSpec((1, 32), lambda i: (0, 0)),
            pl.BlockSpec((1, 1), lambda i: (0, 0)),
        ],
        out_specs=pl.BlockSpec((BLK, 1), lambda i: (i, 0)),
        out_shape=jax.ShapeDtypeStruct((B, 1), jnp.float32),
    )(ce, pe, cb, pb, w1c, w1p,
      b1.reshape(1, 64), W2, b2.reshape(1, 32), W3.reshape(1, 32), const)
    return out.reshape(B)


# speedup vs baseline: 2.7663x; 2.7663x over previous
import jax
import jax.numpy as jnp
from jax import lax
from jax.experimental import pallas as pl
from jax.experimental.pallas import tpu as pltpu
from jax.experimental.pallas import tpu_sc as plsc

B = 16384
D = 32
NC = 2
NS = 16
NW = NC * NS
BPW = B // NW
G = 16
NG = BPW // G

BLK = 2048


def _gather_one_table(idx_v, tabT, win_v, stage_v, sem):
    def group_body(g, _):
        idv = idx_v[pl.ds(g * G, G)]
        for l in range(G):
            w0 = pl.multiple_of((idv[l] >> 7) * 128, 128)
            pltpu.async_copy(tabT.at[:, pl.ds(w0, 128)], win_v.at[l], sem)
        for l in range(G):
            pltpu.make_async_copy(tabT.at[:, pl.ds(0, 128)],
                                  win_v.at[l], sem).wait()
        r0 = lax.iota(jnp.int32, G)
        for l in range(G):
            col = jnp.full((G,), idv[l] & 127, jnp.int32)
            g0 = plsc.load_gather(win_v.at[l], [r0, col])
            g1 = plsc.load_gather(win_v.at[l], [r0 + G, col])
            j = (g * G + l) * D
            stage_v[pl.ds(j, 16)] = g0
            stage_v[pl.ds(j + 16, 16)] = g1
        return 0

    lax.fori_loop(0, NG, group_body, 0)


def _sc_gather(cust_ids, prod_ids, cembT, pembT, cbias, pbias,
               ce_out, pe_out, cb_out, pb_out,
               cidx_v, pidx_v, win_v, stage_v, cb_v, pb_v, sem, bsem):
    wid = lax.axis_index("s") * NC + lax.axis_index("c")
    base = wid * BPW
    pltpu.sync_copy(cust_ids.at[pl.ds(base, BPW)], cidx_v)
    pltpu.sync_copy(prod_ids.at[pl.ds(base, BPW)], pidx_v)

    pltpu.async_copy(cbias.at[cidx_v], cb_v, bsem)
    pltpu.async_copy(pbias.at[pidx_v], pb_v, bsem)

    _gather_one_table(cidx_v, cembT, win_v, stage_v, sem)
    pltpu.sync_copy(stage_v, ce_out.at[pl.ds(base * D, BPW * D)])
    _gather_one_table(pidx_v, pembT, win_v, stage_v, sem)
    pltpu.sync_copy(stage_v, pe_out.at[pl.ds(base * D, BPW * D)])

    pltpu.make_async_copy(cbias.at[pl.ds(0, BPW)], cb_v, bsem).wait()
    pltpu.make_async_copy(pbias.at[pl.ds(0, BPW)], pb_v, bsem).wait()
    pltpu.sync_copy(cb_v, cb_out.at[pl.ds(base, BPW)])
    pltpu.sync_copy(pb_v, pb_out.at[pl.ds(base, BPW)])


def _dense_body(ce_ref, pe_ref, cb_ref, pb_ref, w1c_ref, w1p_ref, b1_ref,
                w2_ref, b2_ref, w3_ref, const_ref, out_ref):
    ce = ce_ref[...]
    pe = pe_ref[...]
    mf = jnp.sum(ce * pe, axis=1, keepdims=True)
    h1 = jnp.maximum(
        jnp.dot(ce, w1c_ref[...], preferred_element_type=jnp.float32)
        + jnp.dot(pe, w1p_ref[...], preferred_element_type=jnp.float32)
        + b1_ref[...], 0.0)
    h2 = jnp.maximum(
        jnp.dot(h1, w2_ref[...], preferred_element_type=jnp.float32)
        + b2_ref[...], 0.0)
    mlp = jnp.sum(h2 * w3_ref[...], axis=1, keepdims=True)
    logit = (0.6 * mf + 0.4 * mlp + cb_ref[...] + pb_ref[...]
             + const_ref[...])
    out_ref[...] = jax.nn.sigmoid(logit)


def kernel(customer_ids, product_ids, cust_emb, prod_emb, cust_bias,
           prod_bias, global_bias, W1, b1, W2, b2, W3, b3):
    cids = customer_ids.astype(jnp.int32)
    pids = product_ids.astype(jnp.int32)
    cembT = cust_emb.T
    pembT = prod_emb.T
    cbias = cust_bias.reshape(-1)
    pbias = prod_bias.reshape(-1)

    mesh = plsc.VectorSubcoreMesh(
        core_axis_name="c", subcore_axis_name="s",
        num_cores=NC, num_subcores=NS)
    sc_call = pl.kernel(
        _sc_gather,
        out_type=[
            jax.ShapeDtypeStruct((B * D,), jnp.float32),
            jax.ShapeDtypeStruct((B * D,), jnp.float32),
            jax.ShapeDtypeStruct((B,), jnp.float32),
            jax.ShapeDtypeStruct((B,), jnp.float32),
        ],
        mesh=mesh,
        scratch_types=[
            pltpu.VMEM((BPW,), jnp.int32),
            pltpu.VMEM((BPW,), jnp.int32),
            pltpu.VMEM((G, D, 128), jnp.float32),
            pltpu.VMEM((BPW * D,), jnp.float32),
            pltpu.VMEM((BPW,), jnp.float32),
            pltpu.VMEM((BPW,), jnp.float32),
            pltpu.SemaphoreType.DMA,
            pltpu.SemaphoreType.DMA,
        ],
        compiler_params=pltpu.CompilerParams(use_tc_tiling_on_sc=True,
                                             needs_layout_passes=False),
    )
    ce_f, pe_f, cb, pb = sc_call(cids, pids, cembT, pembT, cbias, pbias)
    ce = ce_f.reshape(B, D)
    pe = pe_f.reshape(B, D)

    w1c = W1[:D, :]
    w1p = W1[D:, :]
    const = (0.4 * b3 + global_bias).reshape(1, 1)

    grid = (B // BLK,)
    out = pl.pallas_call(
        _dense_body,
        grid=grid,
        in_specs=[
            pl.BlockSpec((BLK, D), lambda i: (i, 0)),
            pl.BlockSpec((BLK, D), lambda i: (i, 0)),
            pl.BlockSpec((BLK, 1), lambda i: (i, 0)),
            pl.BlockSpec((BLK, 1), lambda i: (i, 0)),
            pl.BlockSpec((D, 64), lambda i: (0, 0)),
            pl.BlockSpec((D, 64), lambda i: (0, 0)),
            pl.BlockSpec((1, 64), lambda i: (0, 0)),
            pl.BlockSpec((64, D), lambda i: (0, 0)),
            pl.BlockSpec((1, D), lambda i: (0, 0)),
            pl.BlockSpec((1, D), lambda i: (0, 0)),
            pl.BlockSpec((1, 1), lambda i: (0, 0)),
        ],
        out_specs=pl.BlockSpec((BLK, 1), lambda i: (i, 0)),
        out_shape=jax.ShapeDtypeStruct((B, 1), jnp.float32),
    )(ce, pe, cb.reshape(B, 1), pb.reshape(B, 1), w1c, w1p,
      b1.reshape(1, 64), W2, b2.reshape(1, D), W3.reshape(1, D), const)
    return out.reshape(B)


# speedup vs baseline: 3.5785x; 1.2936x over previous
import jax
import jax.numpy as jnp
from jax import lax
from jax.experimental import pallas as pl
from jax.experimental.pallas import tpu as pltpu
from jax.experimental.pallas import tpu_sc as plsc

B = 16384
D = 32
NC = 2
NS = 16
NW = NC * NS
BPW = B // NW
G = 8
NG = BPW // G

BLK = 2048


def _gather_one_table(idx_v, tabT, winA, winB, stageT, sem):
    r0 = lax.iota(jnp.int32, 16)

    def issue(idv16, half, buf):
        for l in range(G):
            w0 = pl.multiple_of((idv16[half * G + l] >> 7) * 128, 128)
            pltpu.async_copy(tabT.at[:, pl.ds(w0, 128)], buf.at[l], sem)

    def drain(buf):
        for l in range(G):
            pltpu.make_async_copy(tabT.at[:, pl.ds(0, 128)],
                                  buf.at[l], sem).wait()

    def extract(k, idv16, half, buf):
        for l in range(G):
            colv = jnp.full((16,), idv16[half * G + l] & 127, jnp.int32)
            g0 = plsc.load_gather(buf.at[l], [r0, colv])
            g1 = plsc.load_gather(buf.at[l], [r0 + 16, colv])
            jcol = jnp.full((16,), k * 16 + half * G + l, jnp.int32)
            plsc.store_scatter(stageT, [r0, jcol], g0)
            plsc.store_scatter(stageT, [r0 + 16, jcol], g1)

    issue(idx_v[pl.ds(0, 16)], 0, winA)

    def body(k, _):
        idvk = idx_v[pl.ds(k * 16, 16)]
        issue(idvk, 1, winB)
        drain(winA)
        extract(k, idvk, 0, winA)
        idvn = idx_v[pl.ds(jnp.minimum((k + 1) * 16, BPW - 16), 16)]
        issue(idvn, 0, winA)
        drain(winB)
        extract(k, idvk, 1, winB)
        return 0

    lax.fori_loop(0, BPW // 16, body, 0)
    drain(winA)


def _sc_gather(cust_ids, prod_ids, cembT, pembT, cbias, pbias,
               ceT_out, peT_out, cb_out, pb_out,
               cidx_v, pidx_v, winA, winB, stageT, cb_v, pb_v, sem, bsem):
    wid = lax.axis_index("s") * NC + lax.axis_index("c")
    base = wid * BPW
    pltpu.sync_copy(cust_ids.at[pl.ds(base, BPW)], cidx_v)
    pltpu.sync_copy(prod_ids.at[pl.ds(base, BPW)], pidx_v)

    pltpu.async_copy(cbias.at[cidx_v], cb_v, bsem)
    pltpu.async_copy(pbias.at[pidx_v], pb_v, bsem)

    _gather_one_table(cidx_v, cembT, winA, winB, stageT, sem)
    pltpu.sync_copy(stageT, ceT_out.at[:, pl.ds(base, BPW)])
    _gather_one_table(pidx_v, pembT, winA, winB, stageT, sem)
    pltpu.sync_copy(stageT, peT_out.at[:, pl.ds(base, BPW)])

    pltpu.make_async_copy(cbias.at[pl.ds(0, BPW)], cb_v, bsem).wait()
    pltpu.make_async_copy(pbias.at[pl.ds(0, BPW)], pb_v, bsem).wait()
    pltpu.sync_copy(cb_v, cb_out.at[pl.ds(base, BPW)])
    pltpu.sync_copy(pb_v, pb_out.at[pl.ds(base, BPW)])


def _dense_body(ceT_ref, peT_ref, cb_ref, pb_ref, w1cT_ref, w1pT_ref, b1_ref,
                w2T_ref, b2_ref, w3_ref, const_ref, out_ref):
    ceT = ceT_ref[...]
    peT = peT_ref[...]
    mf = jnp.sum(ceT * peT, axis=0, keepdims=True)
    h1 = jnp.maximum(
        jnp.dot(w1cT_ref[...], ceT, preferred_element_type=jnp.float32)
        + jnp.dot(w1pT_ref[...], peT, preferred_element_type=jnp.float32)
        + b1_ref[...], 0.0)
    h2 = jnp.maximum(
        jnp.dot(w2T_ref[...], h1, preferred_element_type=jnp.float32)
        + b2_ref[...], 0.0)
    mlp = jnp.sum(h2 * w3_ref[...], axis=0, keepdims=True)
    logit = (0.6 * mf + 0.4 * mlp + cb_ref[...] + pb_ref[...]
             + const_ref[...])
    out_ref[...] = jax.nn.sigmoid(logit)


def kernel(customer_ids, product_ids, cust_emb, prod_emb, cust_bias,
           prod_bias, global_bias, W1, b1, W2, b2, W3, b3):
    cids = customer_ids.astype(jnp.int32)
    pids = product_ids.astype(jnp.int32)
    cembT = cust_emb.T
    pembT = prod_emb.T
    cbias = cust_bias.reshape(-1)
    pbias = prod_bias.reshape(-1)

    mesh = plsc.VectorSubcoreMesh(
        core_axis_name="c", subcore_axis_name="s",
        num_cores=NC, num_subcores=NS)
    sc_call = pl.kernel(
        _sc_gather,
        out_type=[
            jax.ShapeDtypeStruct((D, B), jnp.float32),
            jax.ShapeDtypeStruct((D, B), jnp.float32),
            jax.ShapeDtypeStruct((B,), jnp.float32),
            jax.ShapeDtypeStruct((B,), jnp.float32),
        ],
        mesh=mesh,
        scratch_types=[
            pltpu.VMEM((BPW,), jnp.int32),
            pltpu.VMEM((BPW,), jnp.int32),
            pltpu.VMEM((G, D, 128), jnp.float32),
            pltpu.VMEM((G, D, 128), jnp.float32),
            pltpu.VMEM((D, BPW), jnp.float32),
            pltpu.VMEM((BPW,), jnp.float32),
            pltpu.VMEM((BPW,), jnp.float32),
            pltpu.SemaphoreType.DMA,
            pltpu.SemaphoreType.DMA,
        ],
        compiler_params=pltpu.CompilerParams(use_tc_tiling_on_sc=True,
                                             needs_layout_passes=False),
    )
    ceT, peT, cb, pb = sc_call(cids, pids, cembT, pembT, cbias, pbias)

    w1cT = W1[:D, :].T
    w1pT = W1[D:, :].T
    const = (0.4 * b3 + global_bias).reshape(1, 1)

    grid = (B // BLK,)
    out = pl.pallas_call(
        _dense_body,
        grid=grid,
        in_specs=[
            pl.BlockSpec((D, BLK), lambda i: (0, i)),
            pl.BlockSpec((D, BLK), lambda i: (0, i)),
            pl.BlockSpec((1, BLK), lambda i: (0, i)),
            pl.BlockSpec((1, BLK), lambda i: (0, i)),
            pl.BlockSpec((64, D), lambda i: (0, 0)),
            pl.BlockSpec((64, D), lambda i: (0, 0)),
            pl.BlockSpec((64, 1), lambda i: (0, 0)),
            pl.BlockSpec((D, 64), lambda i: (0, 0)),
            pl.BlockSpec((D, 1), lambda i: (0, 0)),
            pl.BlockSpec((D, 1), lambda i: (0, 0)),
            pl.BlockSpec((1, 1), lambda i: (0, 0)),
        ],
        out_specs=pl.BlockSpec((1, BLK), lambda i: (0, i)),
        out_shape=jax.ShapeDtypeStruct((1, B), jnp.float32),
    )(ceT, peT, cb.reshape(1, B), pb.reshape(1, B), w1cT, w1pT,
      b1.reshape(64, 1), W2.T, b2.reshape(D, 1), W3, const)
    return out.reshape(B)
